# baseline (speedup 1.0000x reference)
import functools

import jax
import jax.numpy as jnp
from jax import lax
from jax.experimental import pallas as pl
from jax.experimental.pallas import tpu as pltpu
from jax.experimental.pallas import tpu_sc as plsc

_NUM_CORES = 2
_NUM_SUBCORES = 16
_NUM_WORKERS = _NUM_CORES * _NUM_SUBCORES
_L = 16
_W = 128
_C = 256


def _transpose_kernel(v_pad, d):
    mesh = plsc.VectorSubcoreMesh(
        core_axis_name="c", subcore_axis_name="s",
        num_cores=_NUM_CORES, num_subcores=_NUM_SUBCORES)
    n_win = v_pad // _W
    per_w = (n_win + _NUM_WORKERS - 1) // _NUM_WORKERS
    per_w += per_w % 2

    @functools.partial(
        pl.kernel,
        out_type=jax.ShapeDtypeStruct((v_pad // 2, 2 * d), jnp.float32),
        mesh=mesh,
        scratch_types=[
            pltpu.VMEM((2, d, _W), jnp.float32),
            pltpu.VMEM((2, _W // 2, 2 * d + 1), jnp.float32),
            pltpu.SemaphoreType.DMA,
            pltpu.SemaphoreType.DMA,
            pltpu.SemaphoreType.DMA,
            pltpu.SemaphoreType.DMA,
        ],
        compiler_params=pltpu.CompilerParams(needs_layout_passes=False),
    )
    def tkern(tt_hbm, out_hbm, src2, dst2, rs0, rs1, ws0, ws1):
        wid = lax.axis_index("s") * _NUM_CORES + lax.axis_index("c")
        lane = lax.iota(jnp.int32, _L)
        rsem = (rs0, rs1)
        wsem = (ws0, ws1)

        def rd(u, b):
            win = u * _NUM_WORKERS + wid

            @pl.when(win < n_win)
            def _():
                pltpu.async_copy(
                    tt_hbm.at[:, pl.ds(win * _W, _W)], src2.at[b], rsem[b])

        rd(0, 0)
        rd(1, 1)

        @pl.loop(0, per_w, step=2)
        def _g(u0):
            for b in range(2):
                u = u0 + b
                win = u * _NUM_WORKERS + wid

                @pl.when(u >= 2)
                def _():
                    prev = (u - 2) * _NUM_WORKERS + wid

                    @pl.when(prev < n_win)
                    def _():
                        pltpu.make_async_copy(
                            dst2.at[b, :, pl.ds(0, 2 * d)],
                            out_hbm.at[pl.ds(0, _W // 2)],
                            wsem[b]).wait()

                @pl.when(win < n_win)
                def _():
                    pltpu.make_async_copy(
                        tt_hbm.at[:, pl.ds(0, _W)], src2.at[b],
                        rsem[b]).wait()
                    @pl.loop(0, _W // _L)
                    def _rgrp(rb):
                        rv16 = lane + _L * rb
                        prow = jax.lax.shift_right_logical(rv16, 1)
                        bcol = jnp.bitwise_and(rv16, 1) * d
                        for cb in range(d // 16):
                            fvs = [jnp.bitwise_and(16 * cb + c + lane, d - 1)
                                   for c in range(16)]
                            vals = [plsc.load_gather(src2.at[b], [fv, rv16])
                                    for fv in fvs]
                            for c in range(16):
                                plsc.store_scatter(
                                    dst2.at[b], [prow, bcol + fvs[c]],
                                    vals[c])
                    pltpu.async_copy(
                        dst2.at[b, :, pl.ds(0, 2 * d)],
                        out_hbm.at[pl.ds(win * (_W // 2), _W // 2)],
                        wsem[b])

                rd(u + 2, b)

        for b in range(2):
            last = (per_w - 2 + b) * _NUM_WORKERS + wid

            @pl.when(last < n_win)
            def _():
                pltpu.make_async_copy(
                    dst2.at[b, :, pl.ds(0, 2 * d)],
                    out_hbm.at[pl.ds(0, _W // 2)],
                    wsem[b]).wait()

    return tkern


def _gather_kernel(v_pad, d, s_planes, b_cols):
    mesh = plsc.VectorSubcoreMesh(
        core_axis_name="c", subcore_axis_name="s",
        num_cores=_NUM_CORES, num_subcores=_NUM_SUBCORES)
    groups = b_cols // _C
    units = s_planes * groups // _NUM_WORKERS

    @functools.partial(
        pl.kernel,
        out_type=jax.ShapeDtypeStruct((s_planes, d, b_cols), jnp.float32),
        mesh=mesh,
        scratch_types=[
            pltpu.VMEM((2, _C), jnp.int32),
            pltpu.VMEM((_C,), jnp.int32),
            pltpu.VMEM((_C,), jnp.int32),
            pltpu.VMEM((2, _C, 2 * d), jnp.float32),
            pltpu.VMEM((2, d, _C + 2), jnp.float32),
            pltpu.SemaphoreType.DMA,
            pltpu.SemaphoreType.DMA,
            pltpu.SemaphoreType.DMA,
            pltpu.SemaphoreType.DMA,
        ],
        compiler_params=pltpu.CompilerParams(needs_layout_passes=False),
    )
    def gkern(tp_hbm, xs_hbm, out_hbm, idx2, pair_a, pair_b, rows2, asm2,
              gs0, gs1, ws0, ws1):
        pair2 = (pair_a, pair_b)
        wid = lax.axis_index("s") * _NUM_CORES + lax.axis_index("c")
        lane = lax.iota(jnp.int32, _L)
        gsem = (gs0, gs1)
        wsem = (ws0, ws1)
        sub = wid % groups
        s_off = wid // groups
        s_step = _NUM_WORKERS // groups

        def issue(u, b):
            @pl.when(u < units)
            def _():
                s = u * s_step + s_off
                off = s * b_cols + sub * _C
                pltpu.sync_copy(xs_hbm.at[pl.ds(off, _C)], idx2.at[b])
                for j in range(_C // _L):
                    iv = idx2[b, pl.ds(_L * j, _L)]
                    pair2[b][pl.ds(_L * j, _L)] = (
                        jax.lax.shift_right_logical(iv, 1))
                pltpu.async_copy(
                    tp_hbm.at[pair2[b]], rows2.at[b], gsem[b])

        issue(0, 0)
        issue(1, 1)

        @pl.loop(0, units, step=2)
        def _g(u0):
            for b in range(2):
                u = u0 + b
                s = u * s_step + s_off
                pltpu.make_async_copy(
                    tp_hbm.at[pair2[b]], rows2.at[b], gsem[b]).wait()

                @pl.when(u >= 2)
                def _():
                    pltpu.make_async_copy(
                        asm2.at[b, :, pl.ds(0, _C)],
                        out_hbm.at[0, :, pl.ds(sub * _C, _C)],
                        wsem[b]).wait()

                @pl.loop(0, _C // _L)
                def _jgrp(j):
                    rv = lane + _L * j
                    qv = jnp.bitwise_and(idx2[b, pl.ds(_L * j, _L)], 1) * d
                    for cb in range(d // 16):
                        fvs = [jnp.bitwise_and(16 * cb + c + lane, d - 1)
                               for c in range(16)]
                        vals = [plsc.load_gather(rows2.at[b], [rv, qv + fv])
                                for fv in fvs]
                        for c in range(16):
                            plsc.store_scatter(
                                asm2.at[b], [fvs[c], rv], vals[c])
                pltpu.async_copy(
                    asm2.at[b, :, pl.ds(0, _C)],
                    out_hbm.at[s, :, pl.ds(sub * _C, _C)],
                    wsem[b])
                issue(u + 2, b)

        for b in range(2):
            pltpu.make_async_copy(
                asm2.at[b, :, pl.ds(0, _C)],
                out_hbm.at[0, :, pl.ds(sub * _C, _C)],
                wsem[b]).wait()

    return gkern


@functools.partial(jax.jit, static_argnums=(2, 3, 4))
def _embed(tt, xs, v_pad, s_planes, b_cols):
    d = tt.shape[0]
    tp = _transpose_kernel(v_pad, d)(tt)
    return _gather_kernel(v_pad, d, s_planes, b_cols)(tp, xs)


def kernel(x, table):
    b, s = x.shape
    v, d = table.shape
    v_pad = ((v + 127) // 128) * 128
    tt = table.T
    xs = x.T.reshape(b * s).astype(jnp.int32)
    out3 = _embed(tt, xs, v_pad, s, b)
    return jnp.transpose(out3, (2, 0, 1))


# speedup vs baseline: 1.0498x; 1.0498x over previous
import functools

import jax
import jax.numpy as jnp
from jax import lax
from jax.experimental import pallas as pl
from jax.experimental.pallas import tpu as pltpu
from jax.experimental.pallas import tpu_sc as plsc

_NUM_CORES = 2
_NUM_SUBCORES = 16
_NUM_WORKERS = _NUM_CORES * _NUM_SUBCORES
_L = 16
_W = 128
_C = 256


def _transpose_kernel(v_pad, d):
    mesh = plsc.VectorSubcoreMesh(
        core_axis_name="c", subcore_axis_name="s",
        num_cores=_NUM_CORES, num_subcores=_NUM_SUBCORES)
    n_win = v_pad // _W
    per_w = (n_win + _NUM_WORKERS - 1) // _NUM_WORKERS
    per_w += per_w % 2

    @functools.partial(
        pl.kernel,
        out_type=jax.ShapeDtypeStruct((v_pad // 2, 2 * d), jnp.float32),
        mesh=mesh,
        scratch_types=[
            pltpu.VMEM((2, d, _W), jnp.float32),
            pltpu.VMEM((2, _W // 2, 2 * d + 1), jnp.float32),
            pltpu.SemaphoreType.DMA,
            pltpu.SemaphoreType.DMA,
            pltpu.SemaphoreType.DMA,
            pltpu.SemaphoreType.DMA,
        ],
        compiler_params=pltpu.CompilerParams(needs_layout_passes=False),
    )
    def tkern(tt_hbm, out_hbm, src2, dst2, rs0, rs1, ws0, ws1):
        wid = lax.axis_index("s") * _NUM_CORES + lax.axis_index("c")
        lane = lax.iota(jnp.int32, _L)
        rsem = (rs0, rs1)
        wsem = (ws0, ws1)

        def rd(u, b):
            win = u * _NUM_WORKERS + wid

            @pl.when(win < n_win)
            def _():
                pltpu.async_copy(
                    tt_hbm.at[:, pl.ds(win * _W, _W)], src2.at[b], rsem[b])

        rd(0, 0)
        rd(1, 1)

        @pl.loop(0, per_w, step=2)
        def _g(u0):
            for b in range(2):
                u = u0 + b
                win = u * _NUM_WORKERS + wid

                @pl.when(u >= 2)
                def _():
                    prev = (u - 2) * _NUM_WORKERS + wid

                    @pl.when(prev < n_win)
                    def _():
                        pltpu.make_async_copy(
                            dst2.at[b, :, pl.ds(0, 2 * d)],
                            out_hbm.at[pl.ds(0, _W // 2)],
                            wsem[b]).wait()

                @pl.when(win < n_win)
                def _():
                    pltpu.make_async_copy(
                        tt_hbm.at[:, pl.ds(0, _W)], src2.at[b],
                        rsem[b]).wait()
                    @pl.loop(0, _W // _L)
                    def _rgrp(rb):
                        rv16 = lane + _L * rb
                        prow = jax.lax.shift_right_logical(rv16, 1)
                        bcol = jnp.bitwise_and(rv16, 1) * d
                        for cb in range(d // 8):
                            fvs = [jnp.bitwise_and(8 * cb + c + lane, d - 1)
                                   for c in range(8)]
                            vals = [plsc.load_gather(src2.at[b], [fv, rv16])
                                    for fv in fvs]
                            for c in range(8):
                                plsc.store_scatter(
                                    dst2.at[b], [prow, bcol + fvs[c]],
                                    vals[c])
                    pltpu.async_copy(
                        dst2.at[b, :, pl.ds(0, 2 * d)],
                        out_hbm.at[pl.ds(win * (_W // 2), _W // 2)],
                        wsem[b])

                rd(u + 2, b)

        for b in range(2):
            last = (per_w - 2 + b) * _NUM_WORKERS + wid

            @pl.when(last < n_win)
            def _():
                pltpu.make_async_copy(
                    dst2.at[b, :, pl.ds(0, 2 * d)],
                    out_hbm.at[pl.ds(0, _W // 2)],
                    wsem[b]).wait()

    return tkern


def _gather_kernel(v_pad, d, s_planes, b_cols):
    mesh = plsc.VectorSubcoreMesh(
        core_axis_name="c", subcore_axis_name="s",
        num_cores=_NUM_CORES, num_subcores=_NUM_SUBCORES)
    groups = b_cols // _C
    units = s_planes * groups // _NUM_WORKERS

    @functools.partial(
        pl.kernel,
        out_type=jax.ShapeDtypeStruct((s_planes, d, b_cols), jnp.float32),
        mesh=mesh,
        scratch_types=[
            pltpu.VMEM((2, _C), jnp.int32),
            pltpu.VMEM((_C,), jnp.int32),
            pltpu.VMEM((_C,), jnp.int32),
            pltpu.VMEM((2, _C, 2 * d), jnp.float32),
            pltpu.VMEM((2, d, _C + 2), jnp.float32),
            pltpu.SemaphoreType.DMA,
            pltpu.SemaphoreType.DMA,
            pltpu.SemaphoreType.DMA,
            pltpu.SemaphoreType.DMA,
        ],
        compiler_params=pltpu.CompilerParams(needs_layout_passes=False),
    )
    def gkern(tp_hbm, xs_hbm, out_hbm, idx2, pair_a, pair_b, rows2, asm2,
              gs0, gs1, ws0, ws1):
        pair2 = (pair_a, pair_b)
        wid = lax.axis_index("s") * _NUM_CORES + lax.axis_index("c")
        lane = lax.iota(jnp.int32, _L)
        gsem = (gs0, gs1)
        wsem = (ws0, ws1)
        sub = wid % groups
        s_off = wid // groups
        s_step = _NUM_WORKERS // groups

        def issue(u, b):
            @pl.when(u < units)
            def _():
                s = u * s_step + s_off
                off = s * b_cols + sub * _C
                pltpu.sync_copy(xs_hbm.at[pl.ds(off, _C)], idx2.at[b])
                for j in range(_C // _L):
                    iv = idx2[b, pl.ds(_L * j, _L)]
                    pair2[b][pl.ds(_L * j, _L)] = (
                        jax.lax.shift_right_logical(iv, 1))
                pltpu.async_copy(
                    tp_hbm.at[pair2[b]], rows2.at[b], gsem[b])

        issue(0, 0)
        issue(1, 1)

        @pl.loop(0, units, step=2)
        def _g(u0):
            for b in range(2):
                u = u0 + b
                s = u * s_step + s_off
                pltpu.make_async_copy(
                    tp_hbm.at[pair2[b]], rows2.at[b], gsem[b]).wait()

                @pl.when(u >= 2)
                def _():
                    pltpu.make_async_copy(
                        asm2.at[b, :, pl.ds(0, _C)],
                        out_hbm.at[0, :, pl.ds(sub * _C, _C)],
                        wsem[b]).wait()

                @pl.loop(0, _C // _L)
                def _jgrp(j):
                    rv = lane + _L * j
                    qv = jnp.bitwise_and(idx2[b, pl.ds(_L * j, _L)], 1) * d
                    for cb in range(d // 8):
                        fvs = [jnp.bitwise_and(8 * cb + c + lane, d - 1)
                               for c in range(8)]
                        vals = [plsc.load_gather(rows2.at[b], [rv, qv + fv])
                                for fv in fvs]
                        for c in range(8):
                            plsc.store_scatter(
                                asm2.at[b], [fvs[c], rv], vals[c])
                pltpu.async_copy(
                    asm2.at[b, :, pl.ds(0, _C)],
                    out_hbm.at[s, :, pl.ds(sub * _C, _C)],
                    wsem[b])
                issue(u + 2, b)

        for b in range(2):
            pltpu.make_async_copy(
                asm2.at[b, :, pl.ds(0, _C)],
                out_hbm.at[0, :, pl.ds(sub * _C, _C)],
                wsem[b]).wait()

    return gkern


@functools.partial(jax.jit, static_argnums=(2, 3, 4))
def _embed(tt, xs, v_pad, s_planes, b_cols):
    d = tt.shape[0]
    tp = _transpose_kernel(v_pad, d)(tt)
    return _gather_kernel(v_pad, d, s_planes, b_cols)(tp, xs)


def kernel(x, table):
    b, s = x.shape
    v, d = table.shape
    v_pad = ((v + 127) // 128) * 128
    tt = table.T
    xs = x.T.reshape(b * s).astype(jnp.int32)
    out3 = _embed(tt, xs, v_pad, s, b)
    return jnp.transpose(out3, (2, 0, 1))
